# baseline (speedup 1.0000x reference)
import jax
import jax.numpy as jnp
from jax import lax
from jax.experimental import pallas as pl
from jax.experimental.pallas import tpu as pltpu
from jax.experimental.pallas import tpu_sc as plsc

B = 1024
L = 200
NUM_ITEMS = 100000
PQ_M = 8
EMB = 128
SUB = EMB // PQ_M
BYTES = EMB // SUB
VALS = 256

NTOK = B * L
NW = 32
TOK_PER_TILE = NTOK // NW
GTOK = 256
NGROUP = TOK_PER_TILE // GTOK
GROWS = GTOK * BYTES

CL_A = 6272
CL_B = NUM_ITEMS - 15 * CL_A


def _sc_body(ids_hbm, codesT_hbm, zeros_hbm, cent_hbm, out_hbm,
             ids_v, pack_st, pk_flat, idx1_v, cpk_v, idx2_v, rows_v,
             cent_sp, packed_sp,
             sem_pk, sem_rows, sem_out):
    sid = lax.axis_index("s")
    wid = sid * 2 + lax.axis_index("c")

    iota = lax.iota(jnp.int32, 16)

    i0 = sid * CL_A

    def repack(cl):
        cl2 = cl // 2
        niter = cl2 // 16
        for sc in (0, 1):
            base = i0 + sc * cl2
            for half in (0, 1):
                for jj in range(4):
                    pltpu.sync_copy(
                        codesT_hbm.at[4 * half + jj].at[pl.ds(base, cl2)],
                        pack_st.at[jj].at[pl.ds(0, cl2)])

                def body(i, _):
                    c0 = pack_st[0, pl.ds(16 * i, 16)]
                    c1 = pack_st[1, pl.ds(16 * i, 16)]
                    c2 = pack_st[2, pl.ds(16 * i, 16)]
                    c3 = pack_st[3, pl.ds(16 * i, 16)]
                    w = (c0 | lax.shift_left(c1, 8) | lax.shift_left(c2, 16)
                         | lax.shift_left(c3, 24))
                    plsc.store_scatter(pk_flat, [32 * i + 2 * iota + half], w)
                    return 0

                lax.fori_loop(0, niter, body, 0, unroll=2)

            pltpu.sync_copy(pk_flat.at[pl.ds(0, 2 * cl2)],
                            packed_sp.at[pl.ds(2 * base, 2 * cl2)])

    @pl.when(sid < 15)
    def _():
        repack(CL_A)

    @pl.when(sid == 15)
    def _():
        repack(CL_B)

    @pl.when(sid == 1)
    def _():
        pltpu.sync_copy(cent_hbm, cent_sp)

    @pl.when(sid == 2)
    def _():
        pltpu.sync_copy(zeros_hbm, packed_sp.at[pl.ds(2 * NUM_ITEMS, 16)])

    pltpu.sync_copy(ids_hbm.at[pl.ds(wid * TOK_PER_TILE, TOK_PER_TILE)],
                    ids_v)

    def premask(i, _):
        v = ids_v[pl.ds(16 * i, 16)]
        ids_v[pl.ds(16 * i, 16)] = jnp.where(v == 0, NUM_ITEMS, v)
        return 0

    lax.fori_loop(0, TOK_PER_TILE // 16, premask, 0, unroll=4)
    plsc.subcore_barrier()

    rowpat = lax.shift_right_logical(iota, 3)
    colpat = lax.bitwise_and(iota, 7)
    wordsel = lax.shift_right_logical(colpat, 2)
    shpat = lax.bitwise_and(colpat, 3) * 8
    joff2 = colpat * VALS
    wordbit = lax.bitwise_and(iota, 1)

    def loop_widx(g, p):
        def body(i, _):
            ids16 = plsc.load_gather(
                ids_v, [g * GTOK + 8 * i + lax.shift_right_logical(iota, 1)])
            idx1_v[p, pl.ds(16 * i, 16)] = 2 * ids16 + wordbit
            return 0

        lax.fori_loop(0, 2 * GTOK // 16, body, 0, unroll=4)

    def unpack_loop(g, p):
        def body(i, _):
            tok_loc = 2 * i + rowpat
            w16 = plsc.load_gather(cpk_v.at[p], [2 * tok_loc + wordsel])
            c = lax.shift_right_logical(w16, shpat) & 255
            idx2_v[p, pl.ds(16 * i, 16)] = c + joff2
            return 0

        lax.fori_loop(0, GROWS // 16, body, 0, unroll=4)

    def issue_pk(p):
        pltpu.async_copy(packed_sp.at[idx1_v.at[p]], cpk_v.at[p], sem_pk)

    def wait_pk():
        pltpu.make_async_copy(packed_sp.at[idx1_v.at[0]], cpk_v.at[0],
                              sem_pk).wait()

    def issue_rows(p):
        pltpu.async_copy(cent_sp.at[idx2_v.at[p]], rows_v.at[p], sem_rows)

    def wait_rows():
        pltpu.make_async_copy(cent_sp.at[idx2_v.at[0]], rows_v.at[0],
                              sem_rows).wait()

    def issue_out(g, p):
        base = wid * (TOK_PER_TILE * BYTES) + g * GROWS
        pltpu.async_copy(rows_v.at[p], out_hbm.at[pl.ds(base, GROWS)],
                         sem_out)

    def wait_out():
        pltpu.make_async_copy(rows_v.at[0], out_hbm.at[pl.ds(0, GROWS)],
                              sem_out).wait()

    loop_widx(0, 0)
    issue_pk(0)

    def group(g, _):
        p = g & 1
        wait_pk()
        unpack_loop(g, p)

        @pl.when(g + 1 < NGROUP)
        def _():
            loop_widx(g + 1, 1 - p)
            issue_pk(1 - p)

        @pl.when(g >= 1)
        def _():
            wait_rows()
            issue_out(g - 1, 1 - p)

        @pl.when(g >= 2)
        def _():
            wait_out()

        issue_rows(p)
        return 0

    lax.fori_loop(0, NGROUP, group, 0)

    wait_rows()
    issue_out(NGROUP - 1, (NGROUP - 1) & 1)
    wait_out()
    wait_out()


@jax.jit
def _run(ids1d, codes_t, zeros16, cent_flat):
    mesh = plsc.VectorSubcoreMesh(core_axis_name="c", subcore_axis_name="s")
    f = pl.kernel(
        _sc_body,
        out_type=jax.ShapeDtypeStruct((NTOK * BYTES, SUB), jnp.float32),
        mesh=mesh,
        compiler_params=pltpu.CompilerParams(
            needs_layout_passes=False, use_tc_tiling_on_sc=False),
        scratch_types=[
            pltpu.VMEM((TOK_PER_TILE,), jnp.int32),
            pltpu.VMEM((4, CL_A // 2), jnp.int32),
            pltpu.VMEM((CL_A,), jnp.int32),
            pltpu.VMEM((2, 2 * GTOK), jnp.int32),
            pltpu.VMEM((2, 2 * GTOK), jnp.int32),
            pltpu.VMEM((2, GROWS), jnp.int32),
            pltpu.VMEM((2, GROWS, SUB), jnp.float32),
            pltpu.VMEM_SHARED((BYTES * VALS, SUB), jnp.float32),
            pltpu.VMEM_SHARED((2 * NUM_ITEMS + 16,), jnp.int32),
            pltpu.SemaphoreType.DMA,
            pltpu.SemaphoreType.DMA,
            pltpu.SemaphoreType.DMA,
        ],
    )
    return f(ids1d, codes_t, zeros16, cent_flat)


def kernel(input_ids, item_codes, centroids):
    ids1d = input_ids.astype(jnp.int32).reshape(NTOK)
    codes_t = item_codes.astype(jnp.int32).T
    zeros16 = jnp.zeros((16,), jnp.int32)
    cent_flat = centroids.reshape(BYTES * VALS, SUB)
    out = _run(ids1d, codes_t, zeros16, cent_flat)
    return out.reshape(B, L, EMB)


# speedup vs baseline: 1.0081x; 1.0081x over previous
import jax
import jax.numpy as jnp
from jax import lax
from jax.experimental import pallas as pl
from jax.experimental.pallas import tpu as pltpu
from jax.experimental.pallas import tpu_sc as plsc

B = 1024
L = 200
NUM_ITEMS = 100000
PQ_M = 8
EMB = 128
SUB = EMB // PQ_M
BYTES = EMB // SUB
VALS = 256

NTOK = B * L
NW = 32
TOK_PER_TILE = NTOK // NW
GTOK = 256
NGROUP = TOK_PER_TILE // GTOK
GROWS = GTOK * BYTES

CL_A = 6272
CL_B = NUM_ITEMS - 15 * CL_A


def _sc_body(ids_hbm, codesT_hbm, cent_hbm, out_hbm,
             ids_v, pack_st, pk_flat, idx1_v, cpk_v, idx2_v, rows_v,
             cent_sp, packed_sp,
             sem_pk, sem_rows, sem_out):
    sid = lax.axis_index("s")
    wid = sid * 2 + lax.axis_index("c")

    iota = lax.iota(jnp.int32, 16)

    i0 = sid * CL_A

    def repack(cl):
        cl2 = cl // 2
        niter = cl2 // 16
        for sc in (0, 1):
            base = i0 + sc * cl2
            for half in (0, 1):
                for jj in range(4):
                    pltpu.sync_copy(
                        codesT_hbm.at[4 * half + jj].at[pl.ds(base, cl2)],
                        pack_st.at[jj].at[pl.ds(0, cl2)])

                def body(i, _):
                    c0 = pack_st[0, pl.ds(16 * i, 16)]
                    c1 = pack_st[1, pl.ds(16 * i, 16)]
                    c2 = pack_st[2, pl.ds(16 * i, 16)]
                    c3 = pack_st[3, pl.ds(16 * i, 16)]
                    w = (c0 | lax.shift_left(c1, 8) | lax.shift_left(c2, 16)
                         | lax.shift_left(c3, 24))
                    plsc.store_scatter(pk_flat, [32 * i + 2 * iota + half], w)
                    return 0

                lax.fori_loop(0, niter, body, 0, unroll=2)

            pltpu.sync_copy(pk_flat.at[pl.ds(0, 2 * cl2)],
                            packed_sp.at[pl.ds(2 * base, 2 * cl2)])

    @pl.when(sid < 15)
    def _():
        repack(CL_A)

    @pl.when(sid == 15)
    def _():
        repack(CL_B)

    @pl.when(sid == 1)
    def _():
        pltpu.sync_copy(cent_hbm, cent_sp)

    pltpu.sync_copy(ids_hbm.at[pl.ds(wid * TOK_PER_TILE, TOK_PER_TILE)],
                    ids_v)
    plsc.subcore_barrier()

    rowpat = lax.shift_right_logical(iota, 3)
    colpat = lax.bitwise_and(iota, 7)
    wordsel = lax.shift_right_logical(colpat, 2)
    shpat = lax.bitwise_and(colpat, 3) * 8
    joff2 = colpat * VALS
    wordbit = lax.bitwise_and(iota, 1)

    def loop_widx(g, p):
        def body(i, _):
            ids16 = plsc.load_gather(
                ids_v, [g * GTOK + 8 * i + lax.shift_right_logical(iota, 1)])
            idx1_v[p, pl.ds(16 * i, 16)] = 2 * ids16 + wordbit
            return 0

        lax.fori_loop(0, 2 * GTOK // 16, body, 0, unroll=4)

    def unpack_loop(g, p):
        def body(i, _):
            tok_loc = 2 * i + rowpat
            w16 = plsc.load_gather(cpk_v.at[p], [2 * tok_loc + wordsel])
            c = lax.shift_right_logical(w16, shpat) & 255
            ids16 = plsc.load_gather(ids_v, [g * GTOK + tok_loc])
            c = jnp.where(ids16 == 0, 0, c)
            idx2_v[p, pl.ds(16 * i, 16)] = c + joff2
            return 0

        lax.fori_loop(0, GROWS // 16, body, 0, unroll=4)

    def issue_pk(p):
        pltpu.async_copy(packed_sp.at[idx1_v.at[p]], cpk_v.at[p], sem_pk)

    def wait_pk():
        pltpu.make_async_copy(packed_sp.at[idx1_v.at[0]], cpk_v.at[0],
                              sem_pk).wait()

    def issue_rows(p):
        pltpu.async_copy(cent_sp.at[idx2_v.at[p]], rows_v.at[p], sem_rows)

    def wait_rows():
        pltpu.make_async_copy(cent_sp.at[idx2_v.at[0]], rows_v.at[0],
                              sem_rows).wait()

    def issue_out(g, p):
        base = wid * (TOK_PER_TILE * BYTES) + g * GROWS
        pltpu.async_copy(rows_v.at[p], out_hbm.at[pl.ds(base, GROWS)],
                         sem_out)

    def wait_out():
        pltpu.make_async_copy(rows_v.at[0], out_hbm.at[pl.ds(0, GROWS)],
                              sem_out).wait()

    loop_widx(0, 0)
    issue_pk(0)

    def group(g, _):
        p = g & 1
        wait_pk()
        unpack_loop(g, p)

        @pl.when(g + 1 < NGROUP)
        def _():
            loop_widx(g + 1, 1 - p)
            issue_pk(1 - p)

        @pl.when(g >= 1)
        def _():
            wait_rows()
            issue_out(g - 1, 1 - p)

        @pl.when(g >= 2)
        def _():
            wait_out()

        issue_rows(p)
        return 0

    lax.fori_loop(0, NGROUP, group, 0)

    wait_rows()
    issue_out(NGROUP - 1, (NGROUP - 1) & 1)
    wait_out()
    wait_out()


@jax.jit
def _run(ids1d, codes_t, cent_flat):
    mesh = plsc.VectorSubcoreMesh(core_axis_name="c", subcore_axis_name="s")
    f = pl.kernel(
        _sc_body,
        out_type=jax.ShapeDtypeStruct((NTOK * BYTES, SUB), jnp.float32),
        mesh=mesh,
        compiler_params=pltpu.CompilerParams(
            needs_layout_passes=False, use_tc_tiling_on_sc=False),
        scratch_types=[
            pltpu.VMEM((TOK_PER_TILE,), jnp.int32),
            pltpu.VMEM((4, CL_A // 2), jnp.int32),
            pltpu.VMEM((CL_A,), jnp.int32),
            pltpu.VMEM((2, 2 * GTOK), jnp.int32),
            pltpu.VMEM((2, 2 * GTOK), jnp.int32),
            pltpu.VMEM((2, GROWS), jnp.int32),
            pltpu.VMEM((2, GROWS, SUB), jnp.float32),
            pltpu.VMEM_SHARED((BYTES * VALS, SUB), jnp.float32),
            pltpu.VMEM_SHARED((2 * NUM_ITEMS,), jnp.int32),
            pltpu.SemaphoreType.DMA,
            pltpu.SemaphoreType.DMA,
            pltpu.SemaphoreType.DMA,
        ],
    )
    return f(ids1d, codes_t, cent_flat)


def kernel(input_ids, item_codes, centroids):
    ids1d = input_ids.astype(jnp.int32).reshape(NTOK)
    codes_t = item_codes.astype(jnp.int32).T
    cent_flat = centroids.reshape(BYTES * VALS, SUB)
    out = _run(ids1d, codes_t, cent_flat)
    return out.reshape(B, L, EMB)
